# baseline (speedup 1.0000x reference)
import functools

import jax
import jax.numpy as jnp
from jax import lax
from jax.experimental import pallas as pl
from jax.experimental.pallas import tpu as pltpu
from jax.experimental.pallas import tpu_sc as plsc

N = 10000
E = 320000
D = 128
NC = 2
NS = 16
NW = NC * NS
EPW = 10240
EP = NW * EPW
NP = 10112
ZROWS = NP // NS
C = 128
NCH = EPW // C
G = 40
NGRP = NCH // G
BF = jnp.float32

_mesh = plsc.VectorSubcoreMesh(
    core_axis_name="c", subcore_axis_name="s", num_cores=NC, num_subcores=NS
)


@functools.partial(
    pl.kernel,
    out_type=jax.ShapeDtypeStruct((NC, NP, D), BF),
    mesh=_mesh,
    scratch_types=[
        pltpu.VMEM((G, C), jnp.int32),
        pltpu.VMEM((G, C), jnp.int32),
        pltpu.VMEM((C, D), BF),
        pltpu.VMEM((C, D), BF),
        pltpu.VMEM((16,), jnp.int32),
        pltpu.VMEM_SHARED((NP, D), BF),
        pltpu.SemaphoreType.DMA,
        pltpu.SemaphoreType.DMA,
        pltpu.SemaphoreType.DMA,
        pltpu.SemaphoreType.DMA,
    ],
)
def _sc_pass(y_hbm, srcg_hbm, srcd_hbm, dstp_hbm, zeros_hbm, ones_hbm,
             mode_hbm, p_hbm, idx_s, idx_d, b0, b1, mode_v, acc,
             s0, s1, t0, t1):
    c = lax.axis_index("c")
    s = lax.axis_index("s")
    wid = s * NC + c
    bufs = (b0, b1)
    gsems = (s0, s1)
    ssems = (t0, t1)
    pltpu.sync_copy(mode_hbm, mode_v)
    pltpu.sync_copy(zeros_hbm.at[pl.ds(s * ZROWS, ZROWS)],
                    acc.at[pl.ds(s * ZROWS, ZROWS)])
    plsc.subcore_barrier()
    m = mode_v[...][0]

    def deg_loop(ihbm):
        pltpu.sync_copy(ones_hbm, b0)

        def grp(g, carry):
            pltpu.sync_copy(ihbm.at[wid, pl.ds(g * G, G)], idx_d)
            for j in range(G):
                pltpu.sync_copy(b0, acc.at[idx_d.at[j]], add=True)
            return carry

        lax.fori_loop(0, NGRP, grp, 0)

    @pl.when(m == 0)
    def _():
        deg_loop(srcd_hbm)

    @pl.when(m == 1)
    def _():
        deg_loop(dstp_hbm)

    def _wait_g(i, k):
        pltpu.make_async_copy(y_hbm.at[idx_s.at[k]], bufs[i], gsems[i]).wait()

    def _wait_s(i):
        pltpu.make_async_copy(bufs[i], acc.at[idx_d.at[0]], ssems[i]).wait()

    @pl.when(m >= 2)
    def _():
        def grp(g, carry):
            @pl.when(g > 0)
            def _():
                _wait_s(1)
            pltpu.sync_copy(srcg_hbm.at[wid, pl.ds(g * G, G)], idx_s)
            pltpu.sync_copy(dstp_hbm.at[wid, pl.ds(g * G, G)], idx_d)
            pltpu.async_copy(y_hbm.at[idx_s.at[0]], b0, s0)

            def pair(p, carry2):
                k = p * 2
                _wait_g(0, k)
                pltpu.async_copy(b0, acc.at[idx_d.at[k]], t0, add=True)

                @pl.when(p > 0)
                def _():
                    _wait_s(1)
                pltpu.async_copy(y_hbm.at[idx_s.at[k + 1]], b1, s1)
                _wait_g(1, k + 1)
                pltpu.async_copy(b1, acc.at[idx_d.at[k + 1]], t1, add=True)
                _wait_s(0)

                @pl.when(p < G // 2 - 1)
                def _():
                    pltpu.async_copy(y_hbm.at[idx_s.at[k + 2]], b0, s0)
                return carry2

            lax.fori_loop(0, G // 2, pair, 0)
            return carry

        lax.fori_loop(0, NGRP, grp, 0)
        _wait_s(1)

    plsc.subcore_barrier()
    for j in range((ZROWS + C - 1) // C):
        rows = min(C, ZROWS - j * C)
        off = s * ZROWS + j * C
        pltpu.sync_copy(acc.at[pl.ds(off, rows)], b0.at[pl.ds(0, rows)])
        pltpu.sync_copy(b0.at[pl.ds(0, rows)], p_hbm.at[c, pl.ds(off, rows)])


_BR = 1000
_GRID = N // _BR


def _tc_body(p_ref, x_ref, ns_ref, nd_ref, b_ref, w_ref, fl_ref,
             y_ref, nso_ref, ndo_ref):
    f_degs = fl_ref[0, 0]
    f_degd = fl_ref[0, 1]
    f_x = fl_ref[0, 2]
    f_relu = fl_ref[0, 3]
    f_ns = fl_ref[0, 4]
    p0 = p_ref[0].astype(jnp.float32)
    p1 = p_ref[1].astype(jnp.float32)
    d = p0[:, 0:1] + p1[:, 0:1]
    norm = jnp.where(d > 0, lax.rsqrt(jnp.maximum(d, 1.0)), 0.0)
    ns = jnp.where(f_degs > 0, norm, ns_ref[...])
    nd = jnp.where(f_degd > 0, norm, nd_ref[...])
    nso_ref[...] = ns
    ndo_ref[...] = nd
    h = (p0 + p1) * nd + b_ref[...]
    h = jnp.where(f_relu > 0, jnp.maximum(h, 0.0), h)
    mm = jnp.where(f_x > 0, x_ref[...], h)
    y_ref[...] = (jnp.dot(mm, w_ref[...],
                          preferred_element_type=jnp.float32) * (
        jnp.where(f_ns > 0, ns, 1.0))).astype(BF)


_spec_rows = pl.BlockSpec((_BR, D), lambda i: (i, 0))
_spec_p = pl.BlockSpec((NC, _BR, D), lambda i: (0, i, 0))
_spec_norm = pl.BlockSpec((_BR, 1), lambda i: (i, 0))
_spec_w = pl.BlockSpec((D, D), lambda i: (0, 0))
_spec_b = pl.BlockSpec((1, D), lambda i: (0, 0))
_spec_fl = pl.BlockSpec((1, 8), lambda i: (0, 0))

_tc_step = pl.pallas_call(
    _tc_body, grid=(_GRID,),
    out_shape=(jax.ShapeDtypeStruct((N, D), BF),
               jax.ShapeDtypeStruct((N, 1), jnp.float32),
               jax.ShapeDtypeStruct((N, 1), jnp.float32)),
    in_specs=[_spec_p, _spec_rows, _spec_norm, _spec_norm, _spec_b,
              _spec_w, _spec_fl],
    out_specs=(_spec_rows, _spec_norm, _spec_norm),
)


def kernel(x, edge_index, W1, b1, W2, b2, W3, b3):
    src = edge_index[0].astype(jnp.int32)
    dst = edge_index[1].astype(jnp.int32)
    pad = EP - E
    src_g = jnp.concatenate([src, jnp.zeros((pad,), jnp.int32)]).reshape(NW, NCH, C)
    src_d = jnp.concatenate([src, jnp.full((pad,), N, jnp.int32)]).reshape(NW, NCH, C)
    dst_p = jnp.concatenate([dst, jnp.full((pad,), N, jnp.int32)]).reshape(NW, NCH, C)
    zerosD = jnp.zeros((NP, D), BF)
    onesCD = jnp.ones((C, D), BF)
    zb = jnp.zeros((1, D), jnp.float32)
    eye = jnp.eye(D, dtype=jnp.float32)

    modes = jnp.repeat(jnp.arange(5, dtype=jnp.int32), 16).reshape(5, 16)
    Ws = jnp.stack([eye, W1, W2, W3, eye])
    bs = jnp.stack([zb, zb, b1.reshape(1, D), b2.reshape(1, D),
                    b3.reshape(1, D)])
    fls = jnp.array([
        [1, 0, 0, 0, 0, 0, 0, 0],
        [0, 1, 1, 0, 1, 0, 0, 0],
        [0, 0, 0, 1, 1, 0, 0, 0],
        [0, 0, 0, 1, 1, 0, 0, 0],
        [0, 0, 0, 0, 0, 0, 0, 0],
    ], jnp.float32).reshape(5, 1, 8)

    def step(carry, mwbf):
        yc, ns_c, nd_c = carry
        mode, W, b, fl = mwbf
        p = _sc_pass(yc, src_g, src_d, dst_p, zerosD, onesCD, mode)
        y2, ns2, nd2 = _tc_step(p, x, ns_c, nd_c, b, W, fl)
        return (y2, ns2, nd2), None

    carry0 = (x.astype(BF), jnp.zeros((N, 1), jnp.float32),
              jnp.zeros((N, 1), jnp.float32))
    (y, _, _), _ = lax.scan(step, carry0, (modes, Ws, bs, fls))
    return y


# speedup vs baseline: 1.0503x; 1.0503x over previous
import functools

import jax
import jax.numpy as jnp
from jax import lax
from jax.experimental import pallas as pl
from jax.experimental.pallas import tpu as pltpu
from jax.experimental.pallas import tpu_sc as plsc

N = 10000
E = 320000
D = 128
NC = 2
NS = 16
NW = NC * NS
EPW = 10240
EP = NW * EPW
NP = 10112
ZROWS = NP // NS
C = 64
NCH = EPW // C
G = 40
NGRP = NCH // G

_mesh = plsc.VectorSubcoreMesh(
    core_axis_name="c", subcore_axis_name="s", num_cores=NC, num_subcores=NS
)


@functools.partial(
    pl.kernel,
    out_type=jax.ShapeDtypeStruct((NC, NP, D), jnp.float32),
    mesh=_mesh,
    scratch_types=[
        pltpu.VMEM((G, C), jnp.int32),
        pltpu.VMEM((G, C), jnp.int32),
        pltpu.VMEM((C, D), jnp.float32),
        pltpu.VMEM((C, D), jnp.float32),
        pltpu.VMEM((C, D), jnp.float32),
        pltpu.VMEM((C, D), jnp.float32),
        pltpu.VMEM((16,), jnp.int32),
        pltpu.VMEM_SHARED((NP, D), jnp.float32),
        pltpu.SemaphoreType.DMA,
        pltpu.SemaphoreType.DMA,
        pltpu.SemaphoreType.DMA,
        pltpu.SemaphoreType.DMA,
        pltpu.SemaphoreType.DMA,
        pltpu.SemaphoreType.DMA,
        pltpu.SemaphoreType.DMA,
        pltpu.SemaphoreType.DMA,
    ],
)
def _sc_pass(y_hbm, srcg_hbm, srcd_hbm, dstp_hbm, zeros_hbm, ones_hbm,
             mode_hbm, p_hbm, idx_s, idx_d, b0, b1, b2, b3, mode_v, acc,
             s0, s1, s2, s3, t0, t1, t2, t3):
    c = lax.axis_index("c")
    s = lax.axis_index("s")
    wid = s * NC + c
    bufs = (b0, b1, b2, b3)
    gsems = (s0, s1, s2, s3)
    ssems = (t0, t1, t2, t3)
    pltpu.sync_copy(mode_hbm, mode_v)
    pltpu.sync_copy(zeros_hbm.at[pl.ds(s * ZROWS, ZROWS)],
                    acc.at[pl.ds(s * ZROWS, ZROWS)])
    plsc.subcore_barrier()
    m = mode_v[...][0]

    def deg_loop(ihbm):
        pltpu.sync_copy(ones_hbm, b0)

        def grp(g, carry):
            pltpu.sync_copy(ihbm.at[wid, g], idx_d)
            for j in range(G):
                pltpu.sync_copy(b0, acc.at[idx_d.at[j]], add=True)
            return carry

        lax.fori_loop(0, NGRP, grp, 0)

    @pl.when(m == 0)
    def _():
        deg_loop(srcd_hbm)

    @pl.when(m == 1)
    def _():
        deg_loop(dstp_hbm)

    def _wait_g(i, k):
        pltpu.make_async_copy(y_hbm.at[idx_s.at[k]], bufs[i], gsems[i]).wait()

    def _wait_s(i):
        pltpu.make_async_copy(bufs[i], acc.at[idx_d.at[0]], ssems[i]).wait()

    @pl.when(m >= 2)
    def _():
        def grp(g, carry):
            @pl.when(g > 0)
            def _():
                for i in range(4):
                    _wait_s(i)
            pltpu.sync_copy(srcg_hbm.at[wid, g], idx_s)
            pltpu.sync_copy(dstp_hbm.at[wid, g], idx_d)
            for i in range(2):
                pltpu.async_copy(y_hbm.at[idx_s.at[i]], bufs[i], gsems[i])

            def dpair(dq, carry2):
                k = dq * 4
                @pl.when(dq > 0)
                def _():
                    for i in range(2, 4):
                        _wait_s(i)
                for i in range(2, 4):
                    pltpu.async_copy(y_hbm.at[idx_s.at[k + i]],
                                     bufs[i], gsems[i])
                for i in range(2):
                    _wait_g(i, k + i)
                    pltpu.async_copy(bufs[i], acc.at[idx_d.at[k + i]],
                                     ssems[i], add=True)

                @pl.when(dq < G // 4 - 1)
                def _():
                    for i in range(2):
                        _wait_s(i)
                        pltpu.async_copy(y_hbm.at[idx_s.at[k + 4 + i]],
                                        bufs[i], gsems[i])
                for i in range(2, 4):
                    _wait_g(i, k + i)
                    pltpu.async_copy(bufs[i], acc.at[idx_d.at[k + i]],
                                     ssems[i], add=True)
                return carry2

            lax.fori_loop(0, G // 4, dpair, 0)
            return carry

        lax.fori_loop(0, NGRP, grp, 0)
        for i in range(4):
            _wait_s(i)

    plsc.subcore_barrier()
    for j in range((ZROWS + C - 1) // C):
        rows = min(C, ZROWS - j * C)
        off = s * ZROWS + j * C
        pltpu.sync_copy(acc.at[pl.ds(off, rows)], b0.at[pl.ds(0, rows)])
        pltpu.sync_copy(b0.at[pl.ds(0, rows)], p_hbm.at[c, pl.ds(off, rows)])


_BR = 1000
_GRID = N // _BR


def _tc_body(p_ref, x_ref, ns_ref, nd_ref, b_ref, w_ref, fl_ref,
             y_ref, nso_ref, ndo_ref):
    f_degs = fl_ref[0, 0]
    f_degd = fl_ref[0, 1]
    f_x = fl_ref[0, 2]
    f_relu = fl_ref[0, 3]
    f_ns = fl_ref[0, 4]
    p0 = p_ref[0]
    p1 = p_ref[1]
    d = p0[:, 0:1] + p1[:, 0:1]
    norm = jnp.where(d > 0, lax.rsqrt(jnp.maximum(d, 1.0)), 0.0)
    ns = jnp.where(f_degs > 0, norm, ns_ref[...])
    nd = jnp.where(f_degd > 0, norm, nd_ref[...])
    nso_ref[...] = ns
    ndo_ref[...] = nd
    h = (p0 + p1) * nd + b_ref[...]
    h = jnp.where(f_relu > 0, jnp.maximum(h, 0.0), h)
    mm = jnp.where(f_x > 0, x_ref[...], h)
    y_ref[...] = jnp.dot(mm, w_ref[...],
                         preferred_element_type=jnp.float32) * (
        jnp.where(f_ns > 0, ns, 1.0))


_spec_rows = pl.BlockSpec((_BR, D), lambda i: (i, 0))
_spec_p = pl.BlockSpec((NC, _BR, D), lambda i: (0, i, 0))
_spec_norm = pl.BlockSpec((_BR, 1), lambda i: (i, 0))
_spec_w = pl.BlockSpec((D, D), lambda i: (0, 0))
_spec_b = pl.BlockSpec((1, D), lambda i: (0, 0))
_spec_fl = pl.BlockSpec((1, 8), lambda i: (0, 0))

_tc_step = pl.pallas_call(
    _tc_body, grid=(_GRID,),
    out_shape=(jax.ShapeDtypeStruct((N, D), jnp.float32),
               jax.ShapeDtypeStruct((N, 1), jnp.float32),
               jax.ShapeDtypeStruct((N, 1), jnp.float32)),
    in_specs=[_spec_p, _spec_rows, _spec_norm, _spec_norm, _spec_b,
              _spec_w, _spec_fl],
    out_specs=(_spec_rows, _spec_norm, _spec_norm),
)


def kernel(x, edge_index, W1, b1, W2, b2, W3, b3):
    src = edge_index[0].astype(jnp.int32)
    dst = edge_index[1].astype(jnp.int32)
    pad = EP - E
    src_g = jnp.concatenate([src, jnp.zeros((pad,), jnp.int32)]).reshape(NW, NGRP, G, C)
    src_d = jnp.concatenate([src, jnp.full((pad,), N, jnp.int32)]).reshape(NW, NGRP, G, C)
    dst_p = jnp.concatenate([dst, jnp.full((pad,), N, jnp.int32)]).reshape(NW, NGRP, G, C)
    zerosD = jnp.zeros((NP, D), jnp.float32)
    onesCD = jnp.ones((C, D), jnp.float32)
    zb = jnp.zeros((1, D), jnp.float32)
    eye = jnp.eye(D, dtype=jnp.float32)

    modes = jnp.repeat(jnp.arange(5, dtype=jnp.int32), 16).reshape(5, 16)
    Ws = jnp.stack([eye, W1, W2, W3, eye])
    bs = jnp.stack([zb, zb, b1.reshape(1, D), b2.reshape(1, D),
                    b3.reshape(1, D)])
    fls = jnp.array([
        [1, 0, 0, 0, 0, 0, 0, 0],
        [0, 1, 1, 0, 1, 0, 0, 0],
        [0, 0, 0, 1, 1, 0, 0, 0],
        [0, 0, 0, 1, 1, 0, 0, 0],
        [0, 0, 0, 0, 0, 0, 0, 0],
    ], jnp.float32).reshape(5, 1, 8)

    def step(carry, mwbf):
        yc, ns_c, nd_c = carry
        mode, W, b, fl = mwbf
        p = _sc_pass(yc, src_g, src_d, dst_p, zerosD, onesCD, mode)
        y2, ns2, nd2 = _tc_step(p, x, ns_c, nd_c, b, W, fl)
        return (y2, ns2, nd2), None

    carry0 = (x, jnp.zeros((N, 1), jnp.float32), jnp.zeros((N, 1), jnp.float32))
    (y, _, _), _ = lax.scan(step, carry0, (modes, Ws, bs, fls))
    return y


# speedup vs baseline: 1.0980x; 1.0454x over previous
import functools

import jax
import jax.numpy as jnp
from jax import lax
from jax.experimental import pallas as pl
from jax.experimental.pallas import tpu as pltpu
from jax.experimental.pallas import tpu_sc as plsc

N = 10000
E = 320000
D = 128
NC = 2
NS = 16
NW = NC * NS
EPW = 10240
EP = NW * EPW
NP = 10112
ZROWS = NP // NS
C = 64
NCH = EPW // C
G = 5
NGRP = NCH // G

_mesh = plsc.VectorSubcoreMesh(
    core_axis_name="c", subcore_axis_name="s", num_cores=NC, num_subcores=NS
)


@functools.partial(
    pl.kernel,
    out_type=jax.ShapeDtypeStruct((NC, NP, D), jnp.float32),
    mesh=_mesh,
    scratch_types=[
        pltpu.VMEM((G, C), jnp.int32),
        pltpu.VMEM((G, C), jnp.int32),
        pltpu.VMEM((G, C), jnp.int32),
        pltpu.VMEM((C, D), jnp.float32),
        pltpu.VMEM((C, D), jnp.float32),
        pltpu.VMEM((C, D), jnp.float32),
        pltpu.VMEM((C, D), jnp.float32),
        pltpu.VMEM((C, D), jnp.float32),
        pltpu.VMEM((16,), jnp.int32),
        pltpu.VMEM_SHARED((NP, D), jnp.float32),
        pltpu.SemaphoreType.DMA,
        pltpu.SemaphoreType.DMA,
        pltpu.SemaphoreType.DMA,
        pltpu.SemaphoreType.DMA,
        pltpu.SemaphoreType.DMA,
        pltpu.SemaphoreType.DMA,
        pltpu.SemaphoreType.DMA,
        pltpu.SemaphoreType.DMA,
        pltpu.SemaphoreType.DMA,
        pltpu.SemaphoreType.DMA,
    ],
)
def _sc_pass(y_hbm, srcg_hbm, srcd_hbm, dstp_hbm, zeros_hbm, ones_hbm,
             mode_hbm, p_hbm, idx_s, idx_da, idx_db, b0, b1, b2, b3, b4,
             mode_v, acc, s0, s1, s2, s3, s4, t0, t1, t2, t3, t4):
    c = lax.axis_index("c")
    s = lax.axis_index("s")
    wid = s * NC + c
    bufs = (b0, b1, b2, b3, b4)
    gsems = (s0, s1, s2, s3, s4)
    ssems = (t0, t1, t2, t3, t4)
    pltpu.sync_copy(mode_hbm, mode_v)
    pltpu.sync_copy(zeros_hbm.at[pl.ds(s * ZROWS, ZROWS)],
                    acc.at[pl.ds(s * ZROWS, ZROWS)])
    plsc.subcore_barrier()
    m = mode_v[...][0]

    def deg_loop(ihbm):
        pltpu.sync_copy(ones_hbm, b0)

        def grp(g, carry):
            pltpu.sync_copy(ihbm.at[wid, g], idx_da)
            for j in range(G):
                pltpu.sync_copy(b0, acc.at[idx_da.at[j]], add=True)
            return carry

        lax.fori_loop(0, NGRP, grp, 0)

    @pl.when(m == 0)
    def _():
        deg_loop(srcd_hbm)

    @pl.when(m == 1)
    def _():
        deg_loop(dstp_hbm)

    def _wait_g(i, k):
        pltpu.make_async_copy(y_hbm.at[idx_s.at[k]], bufs[i], gsems[i]).wait()

    def _wait_s(i):
        pltpu.make_async_copy(bufs[i], acc.at[idx_da.at[0]], ssems[i]).wait()

    @pl.when(m >= 2)
    def _():
        def one_group(gg, idx_d, first):
            pltpu.sync_copy(srcg_hbm.at[wid, gg], idx_s)
            pltpu.sync_copy(dstp_hbm.at[wid, gg], idx_d)
            for i in range(G):
                if first:
                    @pl.when(gg > 0)
                    def _():
                        _wait_s(i)
                else:
                    _wait_s(i)
                pltpu.async_copy(y_hbm.at[idx_s.at[i]], bufs[i], gsems[i])
            for i in range(G):
                _wait_g(i, i)
                pltpu.async_copy(bufs[i], acc.at[idx_d.at[i]],
                                 ssems[i], add=True)

        def super_grp(j, carry):
            one_group(j * 2, idx_da, True)
            one_group(j * 2 + 1, idx_db, False)
            return carry

        lax.fori_loop(0, NGRP // 2, super_grp, 0)
        for i in range(G):
            _wait_s(i)

    plsc.subcore_barrier()
    for j in range((ZROWS + C - 1) // C):
        rows = min(C, ZROWS - j * C)
        off = s * ZROWS + j * C
        pltpu.sync_copy(acc.at[pl.ds(off, rows)], b0.at[pl.ds(0, rows)])
        pltpu.sync_copy(b0.at[pl.ds(0, rows)], p_hbm.at[c, pl.ds(off, rows)])


_BR = 1000
_GRID = N // _BR


def _tc_body(p_ref, x_ref, ns_ref, nd_ref, b_ref, w_ref, fl_ref,
             y_ref, nso_ref, ndo_ref):
    f_degs = fl_ref[0, 0]
    f_degd = fl_ref[0, 1]
    f_x = fl_ref[0, 2]
    f_relu = fl_ref[0, 3]
    f_ns = fl_ref[0, 4]
    p0 = p_ref[0]
    p1 = p_ref[1]
    d = p0[:, 0:1] + p1[:, 0:1]
    norm = jnp.where(d > 0, lax.rsqrt(jnp.maximum(d, 1.0)), 0.0)
    ns = jnp.where(f_degs > 0, norm, ns_ref[...])
    nd = jnp.where(f_degd > 0, norm, nd_ref[...])
    nso_ref[...] = ns
    ndo_ref[...] = nd
    h = (p0 + p1) * nd + b_ref[...]
    h = jnp.where(f_relu > 0, jnp.maximum(h, 0.0), h)
    mm = jnp.where(f_x > 0, x_ref[...], h)
    y_ref[...] = jnp.dot(mm, w_ref[...],
                         preferred_element_type=jnp.float32) * (
        jnp.where(f_ns > 0, ns, 1.0))


_spec_rows = pl.BlockSpec((_BR, D), lambda i: (i, 0))
_spec_p = pl.BlockSpec((NC, _BR, D), lambda i: (0, i, 0))
_spec_norm = pl.BlockSpec((_BR, 1), lambda i: (i, 0))
_spec_w = pl.BlockSpec((D, D), lambda i: (0, 0))
_spec_b = pl.BlockSpec((1, D), lambda i: (0, 0))
_spec_fl = pl.BlockSpec((1, 8), lambda i: (0, 0))

_tc_step = pl.pallas_call(
    _tc_body, grid=(_GRID,),
    out_shape=(jax.ShapeDtypeStruct((N, D), jnp.float32),
               jax.ShapeDtypeStruct((N, 1), jnp.float32),
               jax.ShapeDtypeStruct((N, 1), jnp.float32)),
    in_specs=[_spec_p, _spec_rows, _spec_norm, _spec_norm, _spec_b,
              _spec_w, _spec_fl],
    out_specs=(_spec_rows, _spec_norm, _spec_norm),
)


def kernel(x, edge_index, W1, b1, W2, b2, W3, b3):
    src = edge_index[0].astype(jnp.int32)
    dst = edge_index[1].astype(jnp.int32)
    pad = EP - E
    src_g = jnp.concatenate([src, jnp.zeros((pad,), jnp.int32)]).reshape(NW, NGRP, G, C)
    src_d = jnp.concatenate([src, jnp.full((pad,), N, jnp.int32)]).reshape(NW, NGRP, G, C)
    dst_p = jnp.concatenate([dst, jnp.full((pad,), N, jnp.int32)]).reshape(NW, NGRP, G, C)
    zerosD = jnp.zeros((NP, D), jnp.float32)
    onesCD = jnp.ones((C, D), jnp.float32)
    zb = jnp.zeros((1, D), jnp.float32)
    eye = jnp.eye(D, dtype=jnp.float32)

    modes = jnp.repeat(jnp.arange(5, dtype=jnp.int32), 16).reshape(5, 16)
    Ws = jnp.stack([eye, W1, W2, W3, eye])
    bs = jnp.stack([zb, zb, b1.reshape(1, D), b2.reshape(1, D),
                    b3.reshape(1, D)])
    fls = jnp.array([
        [1, 0, 0, 0, 0, 0, 0, 0],
        [0, 1, 1, 0, 1, 0, 0, 0],
        [0, 0, 0, 1, 1, 0, 0, 0],
        [0, 0, 0, 1, 1, 0, 0, 0],
        [0, 0, 0, 0, 0, 0, 0, 0],
    ], jnp.float32).reshape(5, 1, 8)

    def step(carry, mwbf):
        yc, ns_c, nd_c = carry
        mode, W, b, fl = mwbf
        p = _sc_pass(yc, src_g, src_d, dst_p, zerosD, onesCD, mode)
        y2, ns2, nd2 = _tc_step(p, x, ns_c, nd_c, b, W, fl)
        return (y2, ns2, nd2), None

    carry0 = (x, jnp.zeros((N, 1), jnp.float32), jnp.zeros((N, 1), jnp.float32))
    (y, _, _), _ = lax.scan(step, carry0, (modes, Ws, bs, fls))
    return y
